# baseline (speedup 1.0000x reference)
import functools

import jax
import jax.numpy as jnp
from jax import lax
from jax.experimental import pallas as pl
from jax.experimental.pallas import tpu as pltpu
from jax.experimental.pallas import tpu_sc as plsc

N = 100000
D = 128
G = 64
L = 16
NC = 2
NS = 16
NW = NC * NS
SUB = 16
NPAD = 100096
NSUB = NPAD // SUB
CHUNK = 128
NBUF = 6
SEGS_PER_W = G // NW

_mesh = plsc.VectorSubcoreMesh(core_axis_name="c", subcore_axis_name="s")


def _pool_body(x_hbm, batch_hbm, bsub_hbm, out_hbm,
               bsub_v, win_v, buf0, buf1, buf2, buf3, buf4, buf5, stage,
               semw, sem0, sem1, sem2, sem3, sem4, sem5):
    bufs = (buf0, buf1, buf2, buf3, buf4, buf5)
    sems = (sem0, sem1, sem2, sem3, sem4, sem5)
    w = lax.axis_index("c") * NS + lax.axis_index("s")
    iota = lax.iota(jnp.int32, L)

    pltpu.sync_copy(bsub_hbm, bsub_v)
    svec = jnp.minimum(2 * w + iota, G)
    lo = jnp.zeros((L,), jnp.int32)
    hi = jnp.full((L,), NSUB, jnp.int32)
    for _ in range(13):
        active = lo < hi
        mid = (lo + hi) // 2
        vals = plsc.load_gather(bsub_v, [jnp.minimum(mid, NSUB - 1)])
        less = vals < svec
        lo = jnp.where(jnp.logical_and(active, less), mid + 1, lo)
        hi = jnp.where(jnp.logical_and(active, jnp.logical_not(less)), mid, hi)

    wbs = []
    for j in range(SEGS_PER_W + 1):
        p = jnp.sum(jnp.where(iota == j, lo, 0))
        wb = SUB * jnp.maximum(p - 1, 0)
        pltpu.async_copy(batch_hbm.at[pl.ds(pl.multiple_of(wb, 8), SUB)],
                         win_v.at[j], semw)
        wbs.append(wb)
    bounds = []
    for j in range(SEGS_PER_W + 1):
        pltpu.make_async_copy(batch_hbm.at[pl.ds(0, SUB)], win_v.at[j],
                              semw).wait()
    for j in range(SEGS_PER_W + 1):
        s_j = 2 * w + j
        in_win = jnp.sum(jnp.where(win_v[j] < s_j, 1, 0))
        bounds.append(wbs[j] + in_win)

    for j in range(SEGS_PER_W):
        seg = w * SEGS_PER_W + j
        row_lo = bounds[j]
        row_hi = bounds[j + 1]
        nrows = row_hi - row_lo
        w0 = (row_lo // 8) * 8
        nch = jnp.where(nrows > 0, (row_hi - w0 + CHUNK - 1) // CHUNK, 0)

        def dma_slice(c):
            return x_hbm.at[
                pl.ds(pl.multiple_of(jnp.minimum(w0 + c * CHUNK, N - CHUNK), 8),
                      CHUNK)]

        def start_copy(c, buf, sem):
            pltpu.async_copy(dma_slice(c), buf, sem)

        def wait_copy(c, buf, sem):
            pltpu.make_async_copy(dma_slice(c), buf, sem).wait()

        def process(c, buf, carry):
            wbase = w0 + c * CHUNK
            dma_base = pl.multiple_of(jnp.minimum(wbase, N - CHUNK), 8)
            shift = wbase - dma_base
            r0 = jnp.maximum(row_lo - wbase, 0)
            r1 = jnp.minimum(row_hi - wbase, CHUNK)

            def accum(rr, c2):
                vs = [buf[rr, pl.ds(k * L, L)] for k in range(D // L)]
                mx = tuple(jnp.maximum(c2[k], vs[k]) for k in range(D // L))
                sm = tuple(c2[D // L + k] + vs[k] for k in range(D // L))
                return mx + sm

            def pair_rows(i, c2):
                rr = shift + r0 + 2 * i
                return accum(rr + 1, accum(rr, c2))

            nr = r1 - r0
            carry = lax.fori_loop(0, nr // 2, pair_rows, carry)
            return lax.cond(
                nr % 2 == 1,
                lambda c2: accum(shift + r1 - 1, c2),
                lambda c2: c2, carry)

        for u in range(NBUF - 1):
            @pl.when(u < nch)
            def _(u=u):
                start_copy(u, bufs[u], sems[u])

        def ring_body(jq, carry):
            for u in range(NBUF):
                c = NBUF * jq + u

                def do(cr, c=c, u=u):
                    def issue(cr2, c=c, u=u):
                        v = (u + NBUF - 1) % NBUF
                        start_copy(c + NBUF - 1, bufs[v], sems[v])
                        return cr2

                    cr = lax.cond(c + NBUF - 1 < nch, issue,
                                  lambda cr2: cr2, cr)
                    wait_copy(c, bufs[u], sems[u])
                    return process(c, bufs[u], cr)

                carry = lax.cond(c < nch, do, lambda cr: cr, carry)
            return carry

        init = tuple(jnp.full((L,), -jnp.inf, jnp.float32) for _ in range(D // L)) \
            + tuple(jnp.zeros((L,), jnp.float32) for _ in range(D // L))
        res = lax.fori_loop(0, (nch + NBUF - 1) // NBUF, ring_body, init)

        cnt_vec = jnp.broadcast_to(
            jnp.maximum(nrows, 1).astype(jnp.float32), (L,))
        inv = jnp.ones((L,), jnp.float32) / cnt_vec
        for k in range(D // L):
            stage[pl.ds(k * L, L)] = res[k]
            stage[pl.ds(D + k * L, L)] = res[D // L + k] * inv
        pltpu.sync_copy(stage, out_hbm.at[seg])


_pool_kernel = functools.partial(
    pl.kernel,
    out_type=jax.ShapeDtypeStruct((G, 2 * D), jnp.float32),
    mesh=_mesh,
    compiler_params=pltpu.CompilerParams(needs_layout_passes=False),
    scratch_types=[
        pltpu.VMEM((NSUB,), jnp.int32),
        pltpu.VMEM((SEGS_PER_W + 1, SUB), jnp.int32),
        pltpu.VMEM((CHUNK, D), jnp.float32),
        pltpu.VMEM((CHUNK, D), jnp.float32),
        pltpu.VMEM((CHUNK, D), jnp.float32),
        pltpu.VMEM((CHUNK, D), jnp.float32),
        pltpu.VMEM((CHUNK, D), jnp.float32),
        pltpu.VMEM((CHUNK, D), jnp.float32),
        pltpu.VMEM((2 * D,), jnp.float32),
        pltpu.SemaphoreType.DMA,
        pltpu.SemaphoreType.DMA,
        pltpu.SemaphoreType.DMA,
        pltpu.SemaphoreType.DMA,
        pltpu.SemaphoreType.DMA,
        pltpu.SemaphoreType.DMA,
        pltpu.SemaphoreType.DMA,
    ],
)(_pool_body)


def kernel(x, batch):
    batch = batch.astype(jnp.int32)
    bsub = jnp.concatenate(
        [batch[::SUB], jnp.full((NSUB - N // SUB,), jnp.int32(2**30))])
    return _pool_kernel(x, batch, bsub)


# speedup vs baseline: 1.0320x; 1.0320x over previous
import functools

import jax
import jax.numpy as jnp
from jax import lax
from jax.experimental import pallas as pl
from jax.experimental.pallas import tpu as pltpu
from jax.experimental.pallas import tpu_sc as plsc

N = 100000
D = 128
G = 64
L = 16
NC = 2
NS = 16
NW = NC * NS
SUB = 16
NPAD = 100096
NSUB = NPAD // SUB
CHUNK = 160
NBUF = 4
SEGS_PER_W = G // NW

_mesh = plsc.VectorSubcoreMesh(core_axis_name="c", subcore_axis_name="s")


def _pool_body(x_hbm, batch_hbm, bsub_hbm, out_hbm,
               bsub_v, win_v, buf0, buf1, buf2, buf3, stage,
               semw, sem0, sem1, sem2, sem3):
    bufs = (buf0, buf1, buf2, buf3)
    sems = (sem0, sem1, sem2, sem3)
    w = lax.axis_index("c") * NS + lax.axis_index("s")
    iota = lax.iota(jnp.int32, L)

    pltpu.sync_copy(bsub_hbm, bsub_v)
    svec = jnp.minimum(2 * w + iota, G)
    lo = jnp.zeros((L,), jnp.int32)
    hi = jnp.full((L,), NSUB, jnp.int32)
    for _ in range(13):
        active = lo < hi
        mid = (lo + hi) // 2
        vals = plsc.load_gather(bsub_v, [jnp.minimum(mid, NSUB - 1)])
        less = vals < svec
        lo = jnp.where(jnp.logical_and(active, less), mid + 1, lo)
        hi = jnp.where(jnp.logical_and(active, jnp.logical_not(less)), mid, hi)

    wbs = []
    for j in range(SEGS_PER_W + 1):
        p = jnp.sum(jnp.where(iota == j, lo, 0))
        wb = SUB * jnp.maximum(p - 1, 0)
        pltpu.async_copy(batch_hbm.at[pl.ds(pl.multiple_of(wb, 8), SUB)],
                         win_v.at[j], semw)
        wbs.append(wb)
    bounds = []
    for j in range(SEGS_PER_W + 1):
        pltpu.make_async_copy(batch_hbm.at[pl.ds(0, SUB)], win_v.at[j],
                              semw).wait()
    for j in range(SEGS_PER_W + 1):
        s_j = 2 * w + j
        in_win = jnp.sum(jnp.where(win_v[j] < s_j, 1, 0))
        bounds.append(wbs[j] + in_win)

    for j in range(SEGS_PER_W):
        seg = w * SEGS_PER_W + j
        row_lo = bounds[j]
        row_hi = bounds[j + 1]
        nrows = row_hi - row_lo
        w0 = (row_lo // 8) * 8
        nch = jnp.where(nrows > 0, (row_hi - w0 + CHUNK - 1) // CHUNK, 0)

        def dma_slice(c):
            return x_hbm.at[
                pl.ds(pl.multiple_of(jnp.minimum(w0 + c * CHUNK, N - CHUNK), 8),
                      CHUNK)]

        def start_copy(c, buf, sem):
            pltpu.async_copy(dma_slice(c), buf, sem)

        def wait_copy(c, buf, sem):
            pltpu.make_async_copy(dma_slice(c), buf, sem).wait()

        def process(c, buf, carry):
            wbase = w0 + c * CHUNK
            dma_base = pl.multiple_of(jnp.minimum(wbase, N - CHUNK), 8)
            shift = wbase - dma_base
            r0 = jnp.maximum(row_lo - wbase, 0)
            r1 = jnp.minimum(row_hi - wbase, CHUNK)

            def accum(rr, c2):
                vs = [buf[rr, pl.ds(k * L, L)] for k in range(D // L)]
                mx = tuple(jnp.maximum(c2[k], vs[k]) for k in range(D // L))
                sm = tuple(c2[D // L + k] + vs[k] for k in range(D // L))
                return mx + sm

            def pair_rows(i, c2):
                rr = shift + r0 + 2 * i
                return accum(rr + 1, accum(rr, c2))

            nr = r1 - r0
            carry = lax.fori_loop(0, nr // 2, pair_rows, carry)
            return lax.cond(
                nr % 2 == 1,
                lambda c2: accum(shift + r1 - 1, c2),
                lambda c2: c2, carry)

        for u in range(NBUF - 1):
            @pl.when(u < nch)
            def _(u=u):
                start_copy(u, bufs[u], sems[u])

        def ring_body(jq, carry):
            for u in range(NBUF):
                c = NBUF * jq + u

                def do(cr, c=c, u=u):
                    def issue(cr2, c=c, u=u):
                        v = (u + NBUF - 1) % NBUF
                        start_copy(c + NBUF - 1, bufs[v], sems[v])
                        return cr2

                    cr = lax.cond(c + NBUF - 1 < nch, issue,
                                  lambda cr2: cr2, cr)
                    wait_copy(c, bufs[u], sems[u])
                    return process(c, bufs[u], cr)

                carry = lax.cond(c < nch, do, lambda cr: cr, carry)
            return carry

        init = tuple(jnp.full((L,), -jnp.inf, jnp.float32) for _ in range(D // L)) \
            + tuple(jnp.zeros((L,), jnp.float32) for _ in range(D // L))
        res = lax.fori_loop(0, (nch + NBUF - 1) // NBUF, ring_body, init)

        cnt_vec = jnp.broadcast_to(
            jnp.maximum(nrows, 1).astype(jnp.float32), (L,))
        inv = jnp.ones((L,), jnp.float32) / cnt_vec
        for k in range(D // L):
            stage[pl.ds(k * L, L)] = res[k]
            stage[pl.ds(D + k * L, L)] = res[D // L + k] * inv
        pltpu.sync_copy(stage, out_hbm.at[seg])


_pool_kernel = functools.partial(
    pl.kernel,
    out_type=jax.ShapeDtypeStruct((G, 2 * D), jnp.float32),
    mesh=_mesh,
    compiler_params=pltpu.CompilerParams(needs_layout_passes=False),
    scratch_types=[
        pltpu.VMEM((NSUB,), jnp.int32),
        pltpu.VMEM((SEGS_PER_W + 1, SUB), jnp.int32),
        pltpu.VMEM((CHUNK, D), jnp.float32),
        pltpu.VMEM((CHUNK, D), jnp.float32),
        pltpu.VMEM((CHUNK, D), jnp.float32),
        pltpu.VMEM((CHUNK, D), jnp.float32),
        pltpu.VMEM((2 * D,), jnp.float32),
        pltpu.SemaphoreType.DMA,
        pltpu.SemaphoreType.DMA,
        pltpu.SemaphoreType.DMA,
        pltpu.SemaphoreType.DMA,
        pltpu.SemaphoreType.DMA,
    ],
)(_pool_body)


def kernel(x, batch):
    batch = batch.astype(jnp.int32)
    bsub = jnp.concatenate(
        [batch[::SUB], jnp.full((NSUB - N // SUB,), jnp.int32(2**30))])
    return _pool_kernel(x, batch, bsub)


# speedup vs baseline: 1.0844x; 1.0507x over previous
import functools

import jax
import jax.numpy as jnp
from jax import lax
from jax.experimental import pallas as pl
from jax.experimental.pallas import tpu as pltpu
from jax.experimental.pallas import tpu_sc as plsc

N = 100000
D = 128
G = 64
L = 16
NC = 2
NS = 16
NW = NC * NS
SUB = 16
NPAD = 100096
NSUB = NPAD // SUB
CHUNK = 96
NBUF = 4
SEGS_PER_W = G // NW

_mesh = plsc.VectorSubcoreMesh(core_axis_name="c", subcore_axis_name="s")


def _pool_body(x_hbm, batch_hbm, bsub_hbm, out_hbm,
               bsub_v, win_v, buf0, buf1, buf2, buf3, stage,
               semw, sem0, sem1, sem2, sem3):
    bufs = (buf0, buf1, buf2, buf3)
    sems = (sem0, sem1, sem2, sem3)
    w = lax.axis_index("c") * NS + lax.axis_index("s")
    iota = lax.iota(jnp.int32, L)

    pltpu.sync_copy(bsub_hbm, bsub_v)
    svec = jnp.minimum(2 * w + iota, G)
    lo = jnp.zeros((L,), jnp.int32)
    hi = jnp.full((L,), NSUB, jnp.int32)
    for _ in range(13):
        active = lo < hi
        mid = (lo + hi) // 2
        vals = plsc.load_gather(bsub_v, [jnp.minimum(mid, NSUB - 1)])
        less = vals < svec
        lo = jnp.where(jnp.logical_and(active, less), mid + 1, lo)
        hi = jnp.where(jnp.logical_and(active, jnp.logical_not(less)), mid, hi)

    wbs = []
    for j in range(SEGS_PER_W + 1):
        p = jnp.sum(jnp.where(iota == j, lo, 0))
        wb = SUB * jnp.maximum(p - 1, 0)
        pltpu.async_copy(batch_hbm.at[pl.ds(pl.multiple_of(wb, 8), SUB)],
                         win_v.at[j], semw)
        wbs.append(wb)
    bounds = []
    for j in range(SEGS_PER_W + 1):
        pltpu.make_async_copy(batch_hbm.at[pl.ds(0, SUB)], win_v.at[j],
                              semw).wait()
    for j in range(SEGS_PER_W + 1):
        s_j = 2 * w + j
        in_win = jnp.sum(jnp.where(win_v[j] < s_j, 1, 0))
        bounds.append(wbs[j] + in_win)

    for j in range(SEGS_PER_W):
        seg = w * SEGS_PER_W + j
        row_lo = bounds[j]
        row_hi = bounds[j + 1]
        nrows = row_hi - row_lo
        w0 = (row_lo // 8) * 8
        nch = jnp.where(nrows > 0, (row_hi - w0 + CHUNK - 1) // CHUNK, 0)

        def dma_slice(c):
            return x_hbm.at[
                pl.ds(pl.multiple_of(jnp.minimum(w0 + c * CHUNK, N - CHUNK), 8),
                      CHUNK)]

        def start_copy(c, buf, sem):
            pltpu.async_copy(dma_slice(c), buf, sem)

        def wait_copy(c, buf, sem):
            pltpu.make_async_copy(dma_slice(c), buf, sem).wait()

        def process(c, buf, carry):
            wbase = w0 + c * CHUNK
            dma_base = pl.multiple_of(jnp.minimum(wbase, N - CHUNK), 8)
            shift = wbase - dma_base
            r0 = jnp.maximum(row_lo - wbase, 0)
            r1 = jnp.minimum(row_hi - wbase, CHUNK)

            def accum(rr, c2):
                vs = [buf[rr, pl.ds(k * L, L)] for k in range(D // L)]
                mx = tuple(jnp.maximum(c2[k], vs[k]) for k in range(D // L))
                sm = tuple(c2[D // L + k] + vs[k] for k in range(D // L))
                return mx + sm

            def pair_rows(i, c2):
                rr = shift + r0 + 2 * i
                return accum(rr + 1, accum(rr, c2))

            nr = r1 - r0
            carry = lax.fori_loop(0, nr // 2, pair_rows, carry)
            return lax.cond(
                nr % 2 == 1,
                lambda c2: accum(shift + r1 - 1, c2),
                lambda c2: c2, carry)

        for u in range(NBUF - 1):
            @pl.when(u < nch)
            def _(u=u):
                start_copy(u, bufs[u], sems[u])

        def ring_body(jq, carry):
            for u in range(NBUF):
                c = NBUF * jq + u

                def do(cr, c=c, u=u):
                    def issue(cr2, c=c, u=u):
                        v = (u + NBUF - 1) % NBUF
                        start_copy(c + NBUF - 1, bufs[v], sems[v])
                        return cr2

                    cr = lax.cond(c + NBUF - 1 < nch, issue,
                                  lambda cr2: cr2, cr)
                    wait_copy(c, bufs[u], sems[u])
                    return process(c, bufs[u], cr)

                carry = lax.cond(c < nch, do, lambda cr: cr, carry)
            return carry

        init = tuple(jnp.full((L,), -jnp.inf, jnp.float32) for _ in range(D // L)) \
            + tuple(jnp.zeros((L,), jnp.float32) for _ in range(D // L))
        res = lax.fori_loop(0, (nch + NBUF - 1) // NBUF, ring_body, init)

        cnt_vec = jnp.broadcast_to(
            jnp.maximum(nrows, 1).astype(jnp.float32), (L,))
        inv = jnp.ones((L,), jnp.float32) / cnt_vec
        for k in range(D // L):
            stage[pl.ds(k * L, L)] = res[k]
            stage[pl.ds(D + k * L, L)] = res[D // L + k] * inv
        pltpu.sync_copy(stage, out_hbm.at[seg])


_pool_kernel = functools.partial(
    pl.kernel,
    out_type=jax.ShapeDtypeStruct((G, 2 * D), jnp.float32),
    mesh=_mesh,
    compiler_params=pltpu.CompilerParams(needs_layout_passes=False),
    scratch_types=[
        pltpu.VMEM((NSUB,), jnp.int32),
        pltpu.VMEM((SEGS_PER_W + 1, SUB), jnp.int32),
        pltpu.VMEM((CHUNK, D), jnp.float32),
        pltpu.VMEM((CHUNK, D), jnp.float32),
        pltpu.VMEM((CHUNK, D), jnp.float32),
        pltpu.VMEM((CHUNK, D), jnp.float32),
        pltpu.VMEM((2 * D,), jnp.float32),
        pltpu.SemaphoreType.DMA,
        pltpu.SemaphoreType.DMA,
        pltpu.SemaphoreType.DMA,
        pltpu.SemaphoreType.DMA,
        pltpu.SemaphoreType.DMA,
    ],
)(_pool_body)


def kernel(x, batch):
    batch = batch.astype(jnp.int32)
    bsub = jnp.concatenate(
        [batch[::SUB], jnp.full((NSUB - N // SUB,), jnp.int32(2**30))])
    return _pool_kernel(x, batch, bsub)
